# Initial kernel scaffold; baseline (speedup 1.0000x reference)
import jax
import jax.numpy as jnp
from jax.experimental import pallas as pl


def kernel(seq1, adj, glob_neg_seq, glob_neg_adj, alpha, W_fc, gcn_bias, prelu_a, W_bil, b_bil):
    raise NotImplementedError("write your pallas kernel here")


# speedup vs baseline: 2.1574x; 2.1574x over previous
import jax
import jax.numpy as jnp
from jax.experimental import pallas as pl
from jax.experimental.pallas import tpu as pltpu


def _body(x_ref, adj_ref, gns_ref, wfc_ref, wbil_ref, bias_ref,
          al_ref, pa_ref, bb_ref,
          l1_ref, l2_ref, fix_ref,
          gprev_ref, t0_ref):
    i = pl.program_id(0)
    nblk = pl.num_programs(0)

    x = x_ref[...]
    wfc = wfc_ref[...]
    adjb = adj_ref[...]
    bias = bias_ref[...]
    a = pa_ref[0, 0]
    al = al_ref[0, 0]
    bb = bb_ref[0, 0]

    n_in = wfc.shape[0]
    n_h = wfc.shape[1]
    n = adjb.shape[1] // 4
    assert n == 4

    fts = [jnp.dot(x[:, j * n_in:(j + 1) * n_in], wfc,
                   preferred_element_type=jnp.float32) for j in range(4)]

    def node(r):
        o = (adjb[:, 4 * r + 0:4 * r + 1] * fts[0]
             + adjb[:, 4 * r + 1:4 * r + 2] * fts[1]
             + adjb[:, 4 * r + 2:4 * r + 3] * fts[2]
             + adjb[:, 4 * r + 3:4 * r + 4] * fts[3]) + bias
        return jnp.where(o >= 0.0, o, a * o)

    c = (node(0) + node(1) + node(2)) * (1.0 / 3.0)
    hmv = node(3)

    gns = gns_ref[...]
    negc = (gns[:, 0:n_h] + gns[:, n_h:2 * n_h]
            + gns[:, 2 * n_h:3 * n_h]) * (1.0 / 3.0)

    g = al * c + (1.0 - al) * negc
    t = jnp.dot(hmv, wbil_ref[...], preferred_element_type=jnp.float32)

    l1_ref[...] = jnp.sum(t * g, axis=1, keepdims=True) + bb

    bbk = g.shape[0]
    g_roll = pltpu.roll(g, 1, axis=0)
    row0 = jax.lax.broadcasted_iota(jnp.int32, g.shape, 0) == 0
    g_sh = jnp.where(row0, gprev_ref[...], g_roll)
    l2_ref[...] = jnp.sum(t * g_sh, axis=1, keepdims=True) + bb

    gprev_ref[...] = g[bbk - 1:bbk, :]

    @pl.when(i == 0)
    def _():
        t0_ref[...] = t[0:1, :]

    @pl.when(i == nblk - 1)
    def _():
        fix_ref[...] = jnp.sum(t0_ref[...] * g[bbk - 2:bbk - 1, :],
                               axis=1, keepdims=True) + bb


def kernel(seq1, adj, glob_neg_seq, glob_neg_adj, alpha, W_fc, gcn_bias,
           prelu_a, W_bil, b_bil):
    B, N, N_IN = seq1.shape
    N_H = W_fc.shape[1]
    BB = 2000
    assert B % BB == 0
    G = B // BB

    x = seq1.reshape(B, N * N_IN)
    adj2 = adj.reshape(B, N * N)
    gns = glob_neg_seq.reshape(B, N * N_H)
    wbil = W_bil.reshape(N_H, N_H)
    bias2 = gcn_bias.reshape(1, N_H)
    al2 = alpha.reshape(1, 1)
    pa2 = prelu_a.reshape(1, 1)
    bb2 = b_bil.reshape(1, 1)

    l1, l2, fix = pl.pallas_call(
        _body,
        grid=(G,),
        in_specs=[
            pl.BlockSpec((BB, N * N_IN), lambda i: (i, 0)),
            pl.BlockSpec((BB, N * N), lambda i: (i, 0)),
            pl.BlockSpec((BB, N * N_H), lambda i: (i, 0)),
            pl.BlockSpec((N_IN, N_H), lambda i: (0, 0)),
            pl.BlockSpec((N_H, N_H), lambda i: (0, 0)),
            pl.BlockSpec((1, N_H), lambda i: (0, 0)),
            pl.BlockSpec((1, 1), lambda i: (0, 0)),
            pl.BlockSpec((1, 1), lambda i: (0, 0)),
            pl.BlockSpec((1, 1), lambda i: (0, 0)),
        ],
        out_specs=(
            pl.BlockSpec((BB, 1), lambda i: (i, 0)),
            pl.BlockSpec((BB, 1), lambda i: (i, 0)),
            pl.BlockSpec((1, 1), lambda i: (0, 0)),
        ),
        out_shape=(
            jax.ShapeDtypeStruct((B, 1), jnp.float32),
            jax.ShapeDtypeStruct((B, 1), jnp.float32),
            jax.ShapeDtypeStruct((1, 1), jnp.float32),
        ),
        scratch_shapes=[
            pltpu.VMEM((1, N_H), jnp.float32),
            pltpu.VMEM((1, N_H), jnp.float32),
        ],
        compiler_params=pltpu.CompilerParams(
            dimension_semantics=("arbitrary",),
        ),
    )(x, adj2, gns, W_fc, wbil, bias2, al2, pa2, bb2)

    l2 = l2.at[0, 0].set(fix[0, 0])
    return jnp.concatenate([l1, l2], axis=0)


# speedup vs baseline: 2.5201x; 1.1681x over previous
import numpy as np

import jax
import jax.numpy as jnp
from jax.experimental import pallas as pl
from jax.experimental.pallas import tpu as pltpu


def _body(x_ref, adj_ref, gns_ref, w2_ref, q_ref, ffold_ref, wbil_ref,
          bias_ref, al_ref, pa_ref, bb_ref,
          l1_ref, l2_ref, fix_ref,
          gprev_ref, t0_ref):
    i = pl.program_id(0)
    nblk = pl.num_programs(0)

    n_h = wbil_ref.shape[0]
    x = x_ref[...]
    a = pa_ref[0, 0]
    al = al_ref[0, 0]
    bb = bb_ref[0, 0]
    bias = bias_ref[...]

    w2 = w2_ref[...]
    fp01 = jnp.dot(x[:, 0:256], w2, preferred_element_type=jnp.float32)
    fp23 = jnp.dot(x[:, 256:512], w2, preferred_element_type=jnp.float32)
    fpall = jnp.concatenate([fp01, fp23], axis=1)

    abig = jnp.dot(adj_ref[...], q_ref[...],
                   preferred_element_type=jnp.float32)

    ffold = ffold_ref[...]

    def node(r):
        s = abig[:, 256 * r:256 * (r + 1)] * fpall
        o = jnp.dot(s, ffold, preferred_element_type=jnp.float32) + bias
        return jnp.where(o >= 0.0, o, a * o)

    c = (node(0) + node(1) + node(2)) * (1.0 / 3.0)
    hmv = node(3)

    gns = gns_ref[...]
    negc = (gns[:, 0:n_h] + gns[:, n_h:2 * n_h]
            + gns[:, 2 * n_h:3 * n_h]) * (1.0 / 3.0)

    g = al * c + (1.0 - al) * negc
    t = jnp.dot(hmv, wbil_ref[...], preferred_element_type=jnp.float32)

    l1_ref[...] = jnp.sum(t * g, axis=1, keepdims=True) + bb

    bbk = g.shape[0]
    g_roll = pltpu.roll(g, 1, axis=0)
    row0 = jax.lax.broadcasted_iota(jnp.int32, g.shape, 0) == 0
    g_sh = jnp.where(row0, gprev_ref[...], g_roll)
    l2_ref[...] = jnp.sum(t * g_sh, axis=1, keepdims=True) + bb

    gprev_ref[...] = g[bbk - 1:bbk, :]

    @pl.when(i == 0)
    def _():
        t0_ref[...] = t[0:1, :]

    @pl.when(i == nblk - 1)
    def _():
        fix_ref[...] = jnp.sum(t0_ref[...] * g[bbk - 2:bbk - 1, :],
                               axis=1, keepdims=True) + bb


def kernel(seq1, adj, glob_neg_seq, glob_neg_adj, alpha, W_fc, gcn_bias,
           prelu_a, W_bil, b_bil):
    B, N, N_IN = seq1.shape
    N_H = W_fc.shape[1]
    BB = 2000
    assert B % BB == 0
    G = B // BB

    x = seq1.reshape(B, N * N_IN)
    adj2 = adj.reshape(B, N * N)
    gns = glob_neg_seq.reshape(B, N * N_H)
    wbil = W_bil.reshape(N_H, N_H)
    bias2 = gcn_bias.reshape(1, N_H)
    al2 = alpha.reshape(1, 1)
    pa2 = prelu_a.reshape(1, 1)
    bb2 = b_bil.reshape(1, 1)

    w2 = jnp.zeros((2 * N_IN, 2 * N_H), jnp.float32)
    w2 = w2.at[:N_IN, :N_H].set(W_fc).at[N_IN:, N_H:].set(W_fc)
    k_idx = np.arange(16)[:, None]
    l_idx = np.arange(16 * N_H)[None, :]
    q = jnp.asarray((l_idx // N_H == k_idx).astype(np.float32))
    ffold = jnp.asarray(np.tile(np.eye(N_H, dtype=np.float32), (4, 1)))

    l1, l2, fix = pl.pallas_call(
        _body,
        grid=(G,),
        in_specs=[
            pl.BlockSpec((BB, N * N_IN), lambda i: (i, 0)),
            pl.BlockSpec((BB, N * N), lambda i: (i, 0)),
            pl.BlockSpec((BB, N * N_H), lambda i: (i, 0)),
            pl.BlockSpec((2 * N_IN, 2 * N_H), lambda i: (0, 0)),
            pl.BlockSpec((16, 16 * N_H), lambda i: (0, 0)),
            pl.BlockSpec((4 * N_H, N_H), lambda i: (0, 0)),
            pl.BlockSpec((N_H, N_H), lambda i: (0, 0)),
            pl.BlockSpec((1, N_H), lambda i: (0, 0)),
            pl.BlockSpec((1, 1), lambda i: (0, 0)),
            pl.BlockSpec((1, 1), lambda i: (0, 0)),
            pl.BlockSpec((1, 1), lambda i: (0, 0)),
        ],
        out_specs=(
            pl.BlockSpec((BB, 1), lambda i: (i, 0)),
            pl.BlockSpec((BB, 1), lambda i: (i, 0)),
            pl.BlockSpec((1, 1), lambda i: (0, 0)),
        ),
        out_shape=(
            jax.ShapeDtypeStruct((B, 1), jnp.float32),
            jax.ShapeDtypeStruct((B, 1), jnp.float32),
            jax.ShapeDtypeStruct((1, 1), jnp.float32),
        ),
        scratch_shapes=[
            pltpu.VMEM((1, N_H), jnp.float32),
            pltpu.VMEM((1, N_H), jnp.float32),
        ],
        compiler_params=pltpu.CompilerParams(
            dimension_semantics=("arbitrary",),
        ),
    )(x, adj2, gns, w2, q, ffold, wbil, bias2, al2, pa2, bb2)

    l2 = l2.at[0, 0].set(fix[0, 0])
    return jnp.concatenate([l1, l2], axis=0)
